# baseline (speedup 1.0000x reference)
import functools

import jax
import jax.numpy as jnp
from jax import lax
from jax.experimental import pallas as pl
from jax.experimental.pallas import tpu as pltpu
from jax.experimental.pallas import tpu_sc as plsc

N = 100000
C = 512
NUM_CORES = 2
NUM_SUBCORES = 16
NW = NUM_CORES * NUM_SUBCORES
L = 16

R = 112
G = R // L
LAST_ROW0 = N - R
NUM_CHUNKS = -(-N // R)
MAX_CH_W = -(-NUM_CHUNKS // NW)


def _sc_body(y_hbm, val_hbm, zeros_hbm, out_hbm,
             buf0, buf1, yav, val_v, sem0, sem1, ysem):
    wid = lax.axis_index("s") * NUM_CORES + lax.axis_index("c")
    nch = jnp.where(wid < NUM_CHUNKS % NW, MAX_CH_W, MAX_CH_W - 1)
    bufs = (buf0, buf1)
    sems = (sem0, sem1)

    def row0_of(c):
        return pl.multiple_of(jnp.minimum(c * R, LAST_ROW0), 8)

    pltpu.async_copy(zeros_hbm, buf0, sem0)
    pltpu.async_copy(zeros_hbm, buf1, sem1)

    def y_fire(i, _):
        r0 = row0_of(wid + i * NW)
        pltpu.async_copy(y_hbm.at[pl.ds(r0, R)],
                         yav.at[pl.ds(i * R, R)], ysem)
        return 0
    lax.fori_loop(0, nch, y_fire, 0)

    pltpu.sync_copy(val_hbm, val_v)
    val_vec = val_v[:]
    zero_vec = jnp.zeros((L,), jnp.float32)
    lane = lax.iota(jnp.int32, L)

    def y_drain(i, _):
        pltpu.make_async_copy(y_hbm.at[pl.ds(0, R)],
                              yav.at[pl.ds(0, R)], ysem).wait()
        return 0
    lax.fori_loop(0, nch, y_drain, 0)

    def scatter_groups(buf, ybase, x_vec):
        def g_body(g, _):
            cols = yav[pl.ds(ybase + g * L, L)]
            rows = lane + g * L
            plsc.store_scatter(buf, [rows, cols], x_vec)
            return 0
        lax.fori_loop(0, G, g_body, 0)

    def process(i, buf, sem):
        r0 = row0_of(wid + i * NW)
        pltpu.make_async_copy(buf, out_hbm.at[pl.ds(0, R)], sem).wait()

        @pl.when(i >= 2)
        def _restore():
            scatter_groups(buf, (i - 2) * R, zero_vec)

        scatter_groups(buf, i * R, val_vec)
        pltpu.async_copy(buf, out_hbm.at[pl.ds(r0, R)], sem)

    def pair_body(p, _):
        for b in range(2):
            i = 2 * p + b

            @pl.when(i < nch)
            def _():
                process(i, bufs[b], sems[b])
        return 0
    lax.fori_loop(0, (MAX_CH_W + 1) // 2, pair_body, 0)

    for b in range(2):
        pltpu.make_async_copy(bufs[b], out_hbm.at[pl.ds(0, R)],
                              sems[b]).wait()


_onehot_sc = functools.partial(
    pl.kernel,
    mesh=plsc.VectorSubcoreMesh(core_axis_name="c", subcore_axis_name="s"),
    out_type=jax.ShapeDtypeStruct((N, C), jnp.float32),
    compiler_params=pltpu.CompilerParams(needs_layout_passes=False),
    scratch_types=[
        pltpu.VMEM((R, C), jnp.float32),
        pltpu.VMEM((R, C), jnp.float32),
        pltpu.VMEM((MAX_CH_W * R,), jnp.int32),
        pltpu.VMEM((L,), jnp.float32),
        pltpu.SemaphoreType.DMA,
        pltpu.SemaphoreType.DMA,
        pltpu.SemaphoreType.DMA,
    ],
)(_sc_body)


def kernel(y, val):
    val16 = jnp.broadcast_to(val.astype(jnp.float32), (L,))
    zeros = jnp.zeros((R, C), jnp.float32)
    return _onehot_sc(y, val16, zeros)
